# baseline (speedup 1.0000x reference)
import functools

import jax
import jax.numpy as jnp
from jax import lax
from jax.experimental import pallas as pl
from jax.experimental.pallas import tpu as pltpu
from jax.experimental.pallas import tpu_sc as plsc

N = 100000
E = 1600000
EMB_DIM = 32
HIDDEN = 128
OUT_DIM = 128

NP = 100352
NPF = NP // 8
ROWS_PER_TILE = NP // 16
CROWS_PER_TILE = NP // 128 // 16
CHUNK = 128
BLK = 4
SUB = 7
N_IDX_BLOCKS = 28
E_PAD = 16 * N_IDX_BLOCKS * SUB * BLK * CHUNK
E_ROWS = E_PAD // CHUNK
DEG_BLK = 8
DEG_ROWS_PER_W = E_ROWS // 32
WB_CHUNK = 784
PWB = 128

_MESH = plsc.VectorSubcoreMesh(core_axis_name="c", subcore_axis_name="s")
_SC_PARAMS = pltpu.CompilerParams(use_tc_tiling_on_sc=False,
                                  needs_layout_passes=False)


def _deg_body(dst_hbm, ones_hbm, zeros_hbm, out_hbm,
              dstbuf, onesbuf, dbuf, cbuf, acc, sem):
    c = lax.axis_index("c")
    s = lax.axis_index("s")
    w = c * 16 + s
    pltpu.sync_copy(zeros_hbm, acc.at[pl.ds(s * ROWS_PER_TILE, ROWS_PER_TILE)])
    pltpu.sync_copy(ones_hbm, onesbuf)
    plsc.subcore_barrier()

    def blk(i, carry):
        base = w * DEG_ROWS_PER_W + i * DEG_BLK
        pltpu.sync_copy(dst_hbm.at[pl.ds(base, DEG_BLK)], dstbuf)
        cps = [
            pltpu.make_async_copy(onesbuf, acc.at[dstbuf.at[j]], sem)
            for j in range(DEG_BLK)
        ]
        for cp in cps:
            cp.start(add=True)
        for cp in cps:
            cp.wait()
        return carry

    lax.fori_loop(0, DEG_ROWS_PER_W // DEG_BLK, blk, 0)
    plsc.subcore_barrier()
    iota16 = lax.iota(jnp.int32, 16)
    zeros16 = jnp.zeros((16,), jnp.int32)

    def cchunk(q, carry):
        pltpu.sync_copy(acc.at[pl.ds(s * ROWS_PER_TILE + q * WB_CHUNK,
                                     WB_CHUNK)], dbuf)

        def cgrp(k, carry2):
            v = plsc.load_gather(dbuf, [iota16 + 16 * k, zeros16])
            g = q * (WB_CHUNK // 16) + k
            rowv = jnp.full((16,), g // 8, jnp.int32)
            colv = iota16 + (g % 8) * 16
            plsc.store_scatter(cbuf, [rowv, colv], v)
            return carry2

        lax.fori_loop(0, WB_CHUNK // 16, cgrp, 0)
        return carry

    lax.fori_loop(0, NP // 16 // WB_CHUNK, cchunk, 0)
    pltpu.sync_copy(cbuf, out_hbm.at[c].at[pl.ds(s * CROWS_PER_TILE,
                                                 CROWS_PER_TILE)])


def _deg_call(dst_r, ones, zeros):
    return pl.kernel(
        _deg_body,
        out_type=jax.ShapeDtypeStruct((2, NP // 128, 128), jnp.float32),
        mesh=_MESH,
        scratch_types=[
            pltpu.VMEM((DEG_BLK, CHUNK), jnp.int32),
            pltpu.VMEM((CHUNK, 16), jnp.float32),
            pltpu.VMEM((WB_CHUNK, 16), jnp.float32),
            pltpu.VMEM((CROWS_PER_TILE, 128), jnp.float32),
            pltpu.VMEM_SHARED((NP, 16), jnp.float32),
            pltpu.SemaphoreType.DMA,
        ],
        compiler_params=_SC_PARAMS,
    )(dst_r, ones, zeros)


def _prop_body(n_rounds, slab_hbm, src_hbm, dst_hbm, zeros_hbm, out_hbm,
               srcbuf, dstbuf, rows0, rows1, abuf, tbuf, acc, sem):
    c = lax.axis_index("c")
    s = lax.axis_index("s")
    zsl = pl.ds(s * ROWS_PER_TILE, ROWS_PER_TILE)
    rows = (rows0, rows1)
    for r in range(n_rounds):
        slab_idx = c * n_rounds + r
        pltpu.sync_copy(zeros_hbm, acc.at[zsl])
        plsc.subcore_barrier()

        def gathers(buf_id, j):
            return [
                pltpu.make_async_copy(
                    slab_hbm.at[slab_idx].at[srcbuf.at[j * BLK + k]],
                    rows[buf_id].at[k], sem)
                for k in range(BLK)
            ]

        def blk(i, carry):
            base = s * (N_IDX_BLOCKS * SUB * BLK) + i * (SUB * BLK)
            pltpu.sync_copy(src_hbm.at[pl.ds(base, SUB * BLK)], srcbuf)
            pltpu.sync_copy(dst_hbm.at[pl.ds(base, SUB * BLK)], dstbuf)
            live = gathers(0, 0)
            for cp in live:
                cp.start()
            for j in range(SUB):
                for cp in live:
                    cp.wait()
                if j + 1 < SUB:
                    nxt = gathers((j + 1) % 2, j + 1)
                    for cp in nxt:
                        cp.start()
                else:
                    nxt = []
                for k in range(BLK):
                    pltpu.sync_copy(rows[j % 2].at[k],
                                    acc.at[dstbuf.at[j * BLK + k]], add=True)
                live = nxt
            return carry

        lax.fori_loop(0, N_IDX_BLOCKS, blk, 0)
        plsc.subcore_barrier()

        def wchunk(q, carry):
            base = s * ROWS_PER_TILE + q * PWB
            pltpu.sync_copy(acc.at[pl.ds(base, PWB)], abuf)
            pltpu.sync_copy(slab_hbm.at[slab_idx].at[pl.ds(base, PWB)], tbuf)

            def wrow(i, carry2):
                abuf[i, :] = abuf[i, :] + tbuf[i, :]
                return carry2

            lax.fori_loop(0, PWB, wrow, 0)
            pltpu.sync_copy(abuf, out_hbm.at[slab_idx].at[pl.ds(base, PWB)])
            return carry

        lax.fori_loop(0, ROWS_PER_TILE // PWB, wchunk, 0)
        if r + 1 < n_rounds:
            plsc.subcore_barrier()


def _prop_call(slabs, src_r, dst_r, zeros, n_rounds):
    return pl.kernel(
        functools.partial(_prop_body, n_rounds),
        out_type=jax.ShapeDtypeStruct((2 * n_rounds, NP, 16), jnp.float32),
        mesh=_MESH,
        scratch_types=[
            pltpu.VMEM((SUB * BLK, CHUNK), jnp.int32),
            pltpu.VMEM((SUB * BLK, CHUNK), jnp.int32),
            pltpu.VMEM((BLK, CHUNK, 16), jnp.float32),
            pltpu.VMEM((BLK, CHUNK, 16), jnp.float32),
            pltpu.VMEM((PWB, 16), jnp.float32),
            pltpu.VMEM((PWB, 16), jnp.float32),
            pltpu.VMEM_SHARED((NP, 16), jnp.float32),
            pltpu.SemaphoreType.DMA,
        ],
        compiler_params=_SC_PARAMS,
    )(slabs, src_r, dst_r, zeros)


def _prop_final_body(n_rounds, slab_hbm, src_hbm, dst_hbm, zeros_hbm,
                     dis_hbm, b2_hbm, out_hbm,
                     srcbuf, dstbuf, rows0, rows1, abuf, tbuf, dbuf, b2buf,
                     acc, sem):
    c = lax.axis_index("c")
    s = lax.axis_index("s")
    zsl = pl.ds(s * ROWS_PER_TILE, ROWS_PER_TILE)
    rows = (rows0, rows1)
    pltpu.sync_copy(b2_hbm, b2buf)
    for r in range(n_rounds):
        slab_idx = c * n_rounds + r
        pltpu.sync_copy(zeros_hbm, acc.at[zsl])
        plsc.subcore_barrier()

        def gathers(buf_id, j):
            return [
                pltpu.make_async_copy(
                    slab_hbm.at[slab_idx].at[srcbuf.at[j * BLK + k]],
                    rows[buf_id].at[k], sem)
                for k in range(BLK)
            ]

        def blk(i, carry):
            base = s * (N_IDX_BLOCKS * SUB * BLK) + i * (SUB * BLK)
            pltpu.sync_copy(src_hbm.at[pl.ds(base, SUB * BLK)], srcbuf)
            pltpu.sync_copy(dst_hbm.at[pl.ds(base, SUB * BLK)], dstbuf)
            live = gathers(0, 0)
            for cp in live:
                cp.start()
            for j in range(SUB):
                for cp in live:
                    cp.wait()
                if j + 1 < SUB:
                    nxt = gathers((j + 1) % 2, j + 1)
                    for cp in nxt:
                        cp.start()
                else:
                    nxt = []
                for k in range(BLK):
                    pltpu.sync_copy(rows[j % 2].at[k],
                                    acc.at[dstbuf.at[j * BLK + k]], add=True)
                live = nxt
            return carry

        lax.fori_loop(0, N_IDX_BLOCKS, blk, 0)
        plsc.subcore_barrier()

        b2v = b2buf[slab_idx, :]

        def wchunk(q, carry):
            base = s * ROWS_PER_TILE + q * PWB
            pltpu.sync_copy(acc.at[pl.ds(base, PWB)], abuf)
            pltpu.sync_copy(slab_hbm.at[slab_idx].at[pl.ds(base, PWB)], tbuf)
            pltpu.sync_copy(dis_hbm.at[pl.ds(base, PWB)], dbuf)

            def wgrp(g, carry2):
                for l in range(16):
                    i = g * 16 + l
                    abuf[i, :] = (abuf[i, :] + tbuf[i, :]) * dbuf[i, :] + b2v
                return carry2

            lax.fori_loop(0, PWB // 16, wgrp, 0)
            pltpu.sync_copy(abuf, out_hbm.at[pl.ds(base, PWB), slab_idx, :])
            return carry

        lax.fori_loop(0, ROWS_PER_TILE // PWB, wchunk, 0)
        if r + 1 < n_rounds:
            plsc.subcore_barrier()


def _prop_final_call(slabs, src_r, dst_r, zeros, dis, b2, n_rounds):
    return pl.kernel(
        functools.partial(_prop_final_body, n_rounds),
        out_type=jax.ShapeDtypeStruct((NP, 8, 16), jnp.float32),
        mesh=_MESH,
        scratch_types=[
            pltpu.VMEM((SUB * BLK, CHUNK), jnp.int32),
            pltpu.VMEM((SUB * BLK, CHUNK), jnp.int32),
            pltpu.VMEM((BLK, CHUNK, 16), jnp.float32),
            pltpu.VMEM((BLK, CHUNK, 16), jnp.float32),
            pltpu.VMEM((PWB, 16), jnp.float32),
            pltpu.VMEM((PWB, 16), jnp.float32),
            pltpu.VMEM((PWB, 16), jnp.float32),
            pltpu.VMEM((8, 16), jnp.float32),
            pltpu.VMEM_SHARED((NP, 16), jnp.float32),
            pltpu.SemaphoreType.DMA,
        ],
        compiler_params=_SC_PARAMS,
    )(slabs, src_r, dst_r, zeros, dis, b2)


_TBLK = 1024
_TGRID = NP // _TBLK


def _dis_col(d):
    dT = d.T
    return jnp.concatenate([dT[:, r:r + 1] for r in range(8)], axis=0)


def _fold16(x):
    y = x.reshape(128, 8, 16)
    return jnp.concatenate([y[:, r, :] for r in range(8)], axis=-1)


def _unfold16(x):
    parts = [x[:, 16 * r:16 * (r + 1)] for r in range(8)]
    return jnp.stack(parts, axis=1).reshape(1024, 16)


def _scale_body(dego_ref, emb_ref, dis_ref, slab1_ref, disr_ref):
    deg = dego_ref[0] + dego_ref[1] + 1.0
    dis = lax.rsqrt(deg)
    dis_ref[...] = dis
    disT = _dis_col(dis)
    t1 = disT * emb_ref[...]
    slab1_ref[0] = _fold16(t1[:, :16])
    slab1_ref[1] = _fold16(t1[:, 16:])
    disr_ref[...] = _fold16(jnp.broadcast_to(disT, (_TBLK, 16)))


def _scale_call(dego, emb):
    return pl.pallas_call(
        _scale_body,
        grid=(_TGRID,),
        in_specs=[
            pl.BlockSpec((2, 8, 128), lambda i: (0, i, 0)),
            pl.BlockSpec((_TBLK, EMB_DIM), lambda i: (i, 0)),
        ],
        out_specs=[
            pl.BlockSpec((8, 128), lambda i: (i, 0)),
            pl.BlockSpec((2, 128, 128), lambda i: (0, i, 0)),
            pl.BlockSpec((128, 128), lambda i: (i, 0)),
        ],
        out_shape=[
            jax.ShapeDtypeStruct((NP // 128, 128), jnp.float32),
            jax.ShapeDtypeStruct((2, NPF, 128), jnp.float32),
            jax.ShapeDtypeStruct((NPF, 128), jnp.float32),
        ],
    )(dego, emb)


def _mid_body(dis_ref, u1_ref, W1_ref, b1_ref, W2_ref, slab2_ref):
    disT = _dis_col(dis_ref[...])
    u1 = jnp.concatenate([_unfold16(u1_ref[0]), _unfold16(u1_ref[1])],
                         axis=-1)
    g = disT * u1
    o1 = jnp.dot(g, W1_ref[...], preferred_element_type=jnp.float32) + b1_ref[...]
    h1 = jnp.maximum(o1, 0.0)
    t2 = disT * jnp.dot(h1, W2_ref[...], preferred_element_type=jnp.float32)
    for j in range(8):
        slab2_ref[j] = _fold16(t2[:, 16 * j:16 * (j + 1)])


def _mid_call(dis, u1s, W1, b1, W2):
    return pl.pallas_call(
        _mid_body,
        grid=(_TGRID,),
        in_specs=[
            pl.BlockSpec((8, 128), lambda i: (i, 0)),
            pl.BlockSpec((2, 128, 128), lambda i: (0, i, 0)),
            pl.BlockSpec((EMB_DIM, HIDDEN), lambda i: (0, 0)),
            pl.BlockSpec((1, HIDDEN), lambda i: (0, 0)),
            pl.BlockSpec((HIDDEN, OUT_DIM), lambda i: (0, 0)),
        ],
        out_specs=pl.BlockSpec((8, 128, 128), lambda i: (0, i, 0)),
        out_shape=jax.ShapeDtypeStruct((8, NPF, 128), jnp.float32),
    )(dis, u1s, W1, b1, W2)


def kernel(x, edge_index, emb, W1, b1, W2, b2):
    del x
    src = edge_index[0].astype(jnp.int32)
    dst = edge_index[1].astype(jnp.int32)
    pad = E_PAD - E
    src_r = jnp.concatenate(
        [src, jnp.full((pad,), N, jnp.int32)]).reshape(E_ROWS, CHUNK)
    dst_r = jnp.concatenate(
        [dst, jnp.full((pad,), N, jnp.int32)]).reshape(E_ROWS, CHUNK)
    ones = jnp.ones((CHUNK, 16), jnp.float32)
    zeros = jnp.zeros((ROWS_PER_TILE, 16), jnp.float32)

    dego = _deg_call(dst_r, ones, zeros)
    dis, slab1, disr = _scale_call(dego, emb)
    u1s = _prop_call(slab1.reshape(2, NP, 16), src_r, dst_r, zeros,
                     n_rounds=1)
    slab2 = _mid_call(dis, u1s.reshape(2, NPF, 128), W1,
                      b1.reshape(1, HIDDEN), W2)
    outv = _prop_final_call(slab2.reshape(8, NP, 16), src_r, dst_r, zeros,
                            disr.reshape(NP, 16), b2.reshape(8, 16),
                            n_rounds=4)
    return outv.reshape(NP, OUT_DIM)[:N]


# speedup vs baseline: 1.1983x; 1.1983x over previous
import functools

import jax
import jax.numpy as jnp
from jax import lax
from jax.experimental import pallas as pl
from jax.experimental.pallas import tpu as pltpu
from jax.experimental.pallas import tpu_sc as plsc

N = 100000
E = 1600000
EMB_DIM = 32
HIDDEN = 128
OUT_DIM = 128

NP = 100352
NPF = NP // 8
ROWS_PER_TILE = NP // 16
CROWS_PER_TILE = NP // 128 // 16
CHUNK = 128
BLK = 4
SUB = 7
N_IDX_BLOCKS = 28
E_PAD = 16 * N_IDX_BLOCKS * SUB * BLK * CHUNK
E_ROWS = E_PAD // CHUNK
DEG_BLK = 8
DEG_ROWS_PER_W = E_ROWS // 32
WB_CHUNK = 784
PWB = 128

_MESH = plsc.VectorSubcoreMesh(core_axis_name="c", subcore_axis_name="s")
_SC_PARAMS = pltpu.CompilerParams(use_tc_tiling_on_sc=False,
                                  needs_layout_passes=False)


def _deg_body(dst_hbm, ones_hbm, zeros_hbm, out_hbm,
              dstbuf, onesbuf, dbuf, cbuf, acc, sem):
    c = lax.axis_index("c")
    s = lax.axis_index("s")
    w = c * 16 + s
    pltpu.sync_copy(zeros_hbm, acc.at[pl.ds(s * ROWS_PER_TILE, ROWS_PER_TILE)])
    pltpu.sync_copy(ones_hbm, onesbuf)
    plsc.subcore_barrier()

    def blk(i, carry):
        base = w * DEG_ROWS_PER_W + i * DEG_BLK
        pltpu.sync_copy(dst_hbm.at[pl.ds(base, DEG_BLK)], dstbuf)
        cps = [
            pltpu.make_async_copy(onesbuf, acc.at[dstbuf.at[j]], sem)
            for j in range(DEG_BLK)
        ]
        for cp in cps:
            cp.start(add=True)
        for cp in cps:
            cp.wait()
        return carry

    lax.fori_loop(0, DEG_ROWS_PER_W // DEG_BLK, blk, 0)
    plsc.subcore_barrier()
    iota16 = lax.iota(jnp.int32, 16)
    zeros16 = jnp.zeros((16,), jnp.int32)

    def cchunk(q, carry):
        pltpu.sync_copy(acc.at[pl.ds(s * ROWS_PER_TILE + q * WB_CHUNK,
                                     WB_CHUNK)], dbuf)

        def cgrp(k, carry2):
            v = plsc.load_gather(dbuf, [iota16 + 16 * k, zeros16])
            g = q * (WB_CHUNK // 16) + k
            rowv = jnp.full((16,), g // 8, jnp.int32)
            colv = iota16 + (g % 8) * 16
            plsc.store_scatter(cbuf, [rowv, colv], v)
            return carry2

        lax.fori_loop(0, WB_CHUNK // 16, cgrp, 0)
        return carry

    lax.fori_loop(0, NP // 16 // WB_CHUNK, cchunk, 0)
    pltpu.sync_copy(cbuf, out_hbm.at[c].at[pl.ds(s * CROWS_PER_TILE,
                                                 CROWS_PER_TILE)])


def _deg_call(dst_r, ones, zeros):
    return pl.kernel(
        _deg_body,
        out_type=jax.ShapeDtypeStruct((2, NP // 128, 128), jnp.float32),
        mesh=_MESH,
        scratch_types=[
            pltpu.VMEM((DEG_BLK, CHUNK), jnp.int32),
            pltpu.VMEM((CHUNK, 16), jnp.float32),
            pltpu.VMEM((WB_CHUNK, 16), jnp.float32),
            pltpu.VMEM((CROWS_PER_TILE, 128), jnp.float32),
            pltpu.VMEM_SHARED((NP, 16), jnp.float32),
            pltpu.SemaphoreType.DMA,
        ],
        compiler_params=_SC_PARAMS,
    )(dst_r, ones, zeros)


def _prop_body(n_rounds, slab_hbm, src_hbm, dst_hbm, zeros_hbm, out_hbm,
               srcbuf, dstbuf, rows0, rows1, abuf, tbuf, acc, sem):
    c = lax.axis_index("c")
    s = lax.axis_index("s")
    zsl = pl.ds(s * ROWS_PER_TILE, ROWS_PER_TILE)
    rows = (rows0, rows1)
    for r in range(n_rounds):
        slab_idx = c * n_rounds + r
        pltpu.sync_copy(zeros_hbm, acc.at[zsl])
        plsc.subcore_barrier()

        def gathers(buf_id, j):
            return [
                pltpu.make_async_copy(
                    slab_hbm.at[slab_idx].at[srcbuf.at[j * BLK + k]],
                    rows[buf_id].at[k], sem)
                for k in range(BLK)
            ]

        def blk(i, carry):
            base = s * (N_IDX_BLOCKS * SUB * BLK) + i * (SUB * BLK)
            pltpu.sync_copy(src_hbm.at[pl.ds(base, SUB * BLK)], srcbuf)
            pltpu.sync_copy(dst_hbm.at[pl.ds(base, SUB * BLK)], dstbuf)
            live = gathers(0, 0)
            for cp in live:
                cp.start()
            for j in range(SUB):
                for cp in live:
                    cp.wait()
                if j + 1 < SUB:
                    nxt = gathers((j + 1) % 2, j + 1)
                    for cp in nxt:
                        cp.start()
                else:
                    nxt = []
                for k in range(BLK):
                    pltpu.sync_copy(rows[j % 2].at[k],
                                    acc.at[dstbuf.at[j * BLK + k]], add=True)
                live = nxt
            return carry

        lax.fori_loop(0, N_IDX_BLOCKS, blk, 0)
        plsc.subcore_barrier()

        def wchunk(q, carry):
            base = s * ROWS_PER_TILE + q * PWB
            pltpu.sync_copy(acc.at[pl.ds(base, PWB)], abuf)
            pltpu.sync_copy(slab_hbm.at[slab_idx].at[pl.ds(base, PWB)], tbuf)

            def wrow(i, carry2):
                abuf[i, :] = abuf[i, :] + tbuf[i, :]
                return carry2

            lax.fori_loop(0, PWB, wrow, 0)
            pltpu.sync_copy(abuf, out_hbm.at[slab_idx].at[pl.ds(base, PWB)])
            return carry

        lax.fori_loop(0, ROWS_PER_TILE // PWB, wchunk, 0)
        if r + 1 < n_rounds:
            plsc.subcore_barrier()


def _prop_call(slabs, src_r, dst_r, zeros, n_rounds):
    return pl.kernel(
        functools.partial(_prop_body, n_rounds),
        out_type=jax.ShapeDtypeStruct((2 * n_rounds, NP, 16), jnp.float32),
        mesh=_MESH,
        scratch_types=[
            pltpu.VMEM((SUB * BLK, CHUNK), jnp.int32),
            pltpu.VMEM((SUB * BLK, CHUNK), jnp.int32),
            pltpu.VMEM((BLK, CHUNK, 16), jnp.float32),
            pltpu.VMEM((BLK, CHUNK, 16), jnp.float32),
            pltpu.VMEM((PWB, 16), jnp.float32),
            pltpu.VMEM((PWB, 16), jnp.float32),
            pltpu.VMEM_SHARED((NP, 16), jnp.float32),
            pltpu.SemaphoreType.DMA,
        ],
        compiler_params=_SC_PARAMS,
    )(slabs, src_r, dst_r, zeros)


def _prop_final_body(n_rounds, slab_hbm, src_hbm, dst_hbm, zeros_hbm,
                     dis_hbm, b2_hbm, out_hbm,
                     srcbuf, dstbuf, rows0, rows1, abuf, tbuf, dbuf, b2buf,
                     acc, sem):
    c = lax.axis_index("c")
    s = lax.axis_index("s")
    zsl = pl.ds(s * ROWS_PER_TILE, ROWS_PER_TILE)
    rows = (rows0, rows1)
    pltpu.sync_copy(b2_hbm, b2buf)
    for r in range(n_rounds):
        slab_idx = c * n_rounds + r
        pltpu.sync_copy(zeros_hbm, acc.at[zsl])
        plsc.subcore_barrier()

        def gathers(buf_id, j):
            return [
                pltpu.make_async_copy(
                    slab_hbm.at[slab_idx].at[srcbuf.at[j * BLK + k]],
                    rows[buf_id].at[k], sem)
                for k in range(BLK)
            ]

        def blk(i, carry):
            base = s * (N_IDX_BLOCKS * SUB * BLK) + i * (SUB * BLK)
            pltpu.sync_copy(src_hbm.at[pl.ds(base, SUB * BLK)], srcbuf)
            pltpu.sync_copy(dst_hbm.at[pl.ds(base, SUB * BLK)], dstbuf)
            live = gathers(0, 0)
            for cp in live:
                cp.start()
            for j in range(SUB):
                for cp in live:
                    cp.wait()
                if j + 1 < SUB:
                    nxt = gathers((j + 1) % 2, j + 1)
                    for cp in nxt:
                        cp.start()
                else:
                    nxt = []
                for k in range(BLK):
                    pltpu.sync_copy(rows[j % 2].at[k],
                                    acc.at[dstbuf.at[j * BLK + k]], add=True)
                live = nxt
            return carry

        lax.fori_loop(0, N_IDX_BLOCKS, blk, 0)
        plsc.subcore_barrier()

        b2v = b2buf[slab_idx, :]

        def wchunk(q, carry):
            base = s * ROWS_PER_TILE + q * PWB
            pltpu.sync_copy(acc.at[pl.ds(base, PWB)], abuf)
            pltpu.sync_copy(slab_hbm.at[slab_idx].at[pl.ds(base, PWB)], tbuf)
            pltpu.sync_copy(dis_hbm.at[pl.ds(base, PWB)], dbuf)

            def wgrp(g, carry2):
                for l in range(16):
                    i = g * 16 + l
                    abuf[i, :] = (abuf[i, :] + tbuf[i, :]) * dbuf[i, :] + b2v
                return carry2

            lax.fori_loop(0, PWB // 16, wgrp, 0)
            pltpu.sync_copy(abuf, out_hbm.at[pl.ds(base, PWB),
                                             pl.ds(slab_idx * 16, 16)])
            return carry

        lax.fori_loop(0, ROWS_PER_TILE // PWB, wchunk, 0)
        if r + 1 < n_rounds:
            plsc.subcore_barrier()


def _prop_final_call(slabs, src_r, dst_r, zeros, dis, b2, n_rounds):
    return pl.kernel(
        functools.partial(_prop_final_body, n_rounds),
        out_type=jax.ShapeDtypeStruct((NP, OUT_DIM), jnp.float32),
        mesh=_MESH,
        scratch_types=[
            pltpu.VMEM((SUB * BLK, CHUNK), jnp.int32),
            pltpu.VMEM((SUB * BLK, CHUNK), jnp.int32),
            pltpu.VMEM((BLK, CHUNK, 16), jnp.float32),
            pltpu.VMEM((BLK, CHUNK, 16), jnp.float32),
            pltpu.VMEM((PWB, 16), jnp.float32),
            pltpu.VMEM((PWB, 16), jnp.float32),
            pltpu.VMEM((PWB, 16), jnp.float32),
            pltpu.VMEM((8, 16), jnp.float32),
            pltpu.VMEM_SHARED((NP, 16), jnp.float32),
            pltpu.SemaphoreType.DMA,
        ],
        compiler_params=_SC_PARAMS,
    )(slabs, src_r, dst_r, zeros, dis, b2)


_TBLK = 1024
_TGRID = NP // _TBLK


def _dis_col(d):
    dT = d.T
    return jnp.concatenate([dT[:, r:r + 1] for r in range(8)], axis=0)


def _fold16(x):
    y = x.reshape(128, 8, 16)
    return jnp.concatenate([y[:, r, :] for r in range(8)], axis=-1)


def _unfold16(x):
    parts = [x[:, 16 * r:16 * (r + 1)] for r in range(8)]
    return jnp.stack(parts, axis=1).reshape(1024, 16)


def _scale_body(dego_ref, emb_ref, dis_ref, slab1_ref, disr_ref):
    deg = dego_ref[0] + dego_ref[1] + 1.0
    dis = lax.rsqrt(deg)
    dis_ref[...] = dis
    disT = _dis_col(dis)
    t1 = disT * emb_ref[...]
    slab1_ref[0] = _fold16(t1[:, :16])
    slab1_ref[1] = _fold16(t1[:, 16:])
    disr_ref[...] = _fold16(jnp.broadcast_to(disT, (_TBLK, 16)))


def _scale_call(dego, emb):
    return pl.pallas_call(
        _scale_body,
        grid=(_TGRID,),
        in_specs=[
            pl.BlockSpec((2, 8, 128), lambda i: (0, i, 0)),
            pl.BlockSpec((_TBLK, EMB_DIM), lambda i: (i, 0)),
        ],
        out_specs=[
            pl.BlockSpec((8, 128), lambda i: (i, 0)),
            pl.BlockSpec((2, 128, 128), lambda i: (0, i, 0)),
            pl.BlockSpec((128, 128), lambda i: (i, 0)),
        ],
        out_shape=[
            jax.ShapeDtypeStruct((NP // 128, 128), jnp.float32),
            jax.ShapeDtypeStruct((2, NPF, 128), jnp.float32),
            jax.ShapeDtypeStruct((NPF, 128), jnp.float32),
        ],
    )(dego, emb)


def _mid_body(dis_ref, u1_ref, W1_ref, b1_ref, W2_ref, slab2_ref):
    disT = _dis_col(dis_ref[...])
    u1 = jnp.concatenate([_unfold16(u1_ref[0]), _unfold16(u1_ref[1])],
                         axis=-1)
    g = disT * u1
    o1 = jnp.dot(g, W1_ref[...], preferred_element_type=jnp.float32) + b1_ref[...]
    h1 = jnp.maximum(o1, 0.0)
    t2 = disT * jnp.dot(h1, W2_ref[...], preferred_element_type=jnp.float32)
    for j in range(8):
        slab2_ref[j] = _fold16(t2[:, 16 * j:16 * (j + 1)])


def _mid_call(dis, u1s, W1, b1, W2):
    return pl.pallas_call(
        _mid_body,
        grid=(_TGRID,),
        in_specs=[
            pl.BlockSpec((8, 128), lambda i: (i, 0)),
            pl.BlockSpec((2, 128, 128), lambda i: (0, i, 0)),
            pl.BlockSpec((EMB_DIM, HIDDEN), lambda i: (0, 0)),
            pl.BlockSpec((1, HIDDEN), lambda i: (0, 0)),
            pl.BlockSpec((HIDDEN, OUT_DIM), lambda i: (0, 0)),
        ],
        out_specs=pl.BlockSpec((8, 128, 128), lambda i: (0, i, 0)),
        out_shape=jax.ShapeDtypeStruct((8, NPF, 128), jnp.float32),
    )(dis, u1s, W1, b1, W2)


def kernel(x, edge_index, emb, W1, b1, W2, b2):
    del x
    src = edge_index[0].astype(jnp.int32)
    dst = edge_index[1].astype(jnp.int32)
    pad = E_PAD - E
    src_r = jnp.concatenate(
        [src, jnp.full((pad,), N, jnp.int32)]).reshape(E_ROWS, CHUNK)
    dst_r = jnp.concatenate(
        [dst, jnp.full((pad,), N, jnp.int32)]).reshape(E_ROWS, CHUNK)
    ones = jnp.ones((CHUNK, 16), jnp.float32)
    zeros = jnp.zeros((ROWS_PER_TILE, 16), jnp.float32)

    dego = _deg_call(dst_r, ones, zeros)
    dis, slab1, disr = _scale_call(dego, emb)
    u1s = _prop_call(slab1.reshape(2, NP, 16), src_r, dst_r, zeros,
                     n_rounds=1)
    slab2 = _mid_call(dis, u1s.reshape(2, NPF, 128), W1,
                      b1.reshape(1, HIDDEN), W2)
    outv = _prop_final_call(slab2.reshape(8, NP, 16), src_r, dst_r, zeros,
                            disr.reshape(NP, 16), b2.reshape(8, 16),
                            n_rounds=4)
    return outv[:N]


# speedup vs baseline: 1.2047x; 1.0053x over previous
import functools

import jax
import jax.numpy as jnp
from jax import lax
from jax.experimental import pallas as pl
from jax.experimental.pallas import tpu as pltpu
from jax.experimental.pallas import tpu_sc as plsc

N = 100000
E = 1600000
EMB_DIM = 32
HIDDEN = 128
OUT_DIM = 128

NP = 100352
NPF = NP // 8
ROWS_PER_TILE = NP // 16
CROWS_PER_TILE = NP // 128 // 16
CHUNK = 128
BLK = 4
SUB = 7
N_IDX_BLOCKS = 28
E_PAD = 16 * N_IDX_BLOCKS * SUB * BLK * CHUNK
E_ROWS = E_PAD // CHUNK
DEG_BLK = 8
DEG_ROWS_PER_W = E_ROWS // 32
WB_CHUNK = 784
PWB = 128

_MESH = plsc.VectorSubcoreMesh(core_axis_name="c", subcore_axis_name="s")
_SC_PARAMS = pltpu.CompilerParams(use_tc_tiling_on_sc=False,
                                  needs_layout_passes=False)


def _deg_body(dst_hbm, ones_hbm, zeros_hbm, out_hbm,
              dstbuf, onesbuf, dbuf, cbuf, acc, sem):
    c = lax.axis_index("c")
    s = lax.axis_index("s")
    w = c * 16 + s
    pltpu.sync_copy(zeros_hbm, acc.at[pl.ds(s * ROWS_PER_TILE, ROWS_PER_TILE)])
    pltpu.sync_copy(ones_hbm, onesbuf)
    plsc.subcore_barrier()

    def blk(i, carry):
        base = w * DEG_ROWS_PER_W + i * DEG_BLK
        pltpu.sync_copy(dst_hbm.at[pl.ds(base, DEG_BLK)], dstbuf)
        cps = [
            pltpu.make_async_copy(onesbuf, acc.at[dstbuf.at[j]], sem)
            for j in range(DEG_BLK)
        ]
        for cp in cps:
            cp.start(add=True)
        for cp in cps:
            cp.wait()
        return carry

    lax.fori_loop(0, DEG_ROWS_PER_W // DEG_BLK, blk, 0)
    plsc.subcore_barrier()
    iota16 = lax.iota(jnp.int32, 16)
    zeros16 = jnp.zeros((16,), jnp.int32)

    def cchunk(q, carry):
        pltpu.sync_copy(acc.at[pl.ds(s * ROWS_PER_TILE + q * WB_CHUNK,
                                     WB_CHUNK)], dbuf)

        def cgrp(k, carry2):
            v = plsc.load_gather(dbuf, [iota16 + 16 * k, zeros16])
            g = q * (WB_CHUNK // 16) + k
            rowv = jnp.full((16,), g // 8, jnp.int32)
            colv = iota16 + (g % 8) * 16
            plsc.store_scatter(cbuf, [rowv, colv], v)
            return carry2

        lax.fori_loop(0, WB_CHUNK // 16, cgrp, 0)
        return carry

    lax.fori_loop(0, NP // 16 // WB_CHUNK, cchunk, 0)
    pltpu.sync_copy(cbuf, out_hbm.at[c].at[pl.ds(s * CROWS_PER_TILE,
                                                 CROWS_PER_TILE)])


def _deg_call(dst_r, ones, zeros):
    return pl.kernel(
        _deg_body,
        out_type=jax.ShapeDtypeStruct((2, NP // 128, 128), jnp.float32),
        mesh=_MESH,
        scratch_types=[
            pltpu.VMEM((DEG_BLK, CHUNK), jnp.int32),
            pltpu.VMEM((CHUNK, 16), jnp.float32),
            pltpu.VMEM((WB_CHUNK, 16), jnp.float32),
            pltpu.VMEM((CROWS_PER_TILE, 128), jnp.float32),
            pltpu.VMEM_SHARED((NP, 16), jnp.float32),
            pltpu.SemaphoreType.DMA,
        ],
        compiler_params=_SC_PARAMS,
    )(dst_r, ones, zeros)


def _prop_body(n_rounds, slab_hbm, src_hbm, dst_hbm, zeros_hbm, out_hbm,
               srcbuf, dstbuf, rows0, rows1, abuf, tbuf, acc, sem, sem2):
    c = lax.axis_index("c")
    s = lax.axis_index("s")
    zsl = pl.ds(s * ROWS_PER_TILE, ROWS_PER_TILE)
    rows = (rows0, rows1)
    for r in range(n_rounds):
        slab_idx = c * n_rounds + r
        pltpu.sync_copy(zeros_hbm, acc.at[zsl])
        plsc.subcore_barrier()

        def gathers(buf_id, j):
            return [
                pltpu.make_async_copy(
                    slab_hbm.at[slab_idx].at[srcbuf.at[j * BLK + k]],
                    rows[buf_id].at[k], sem)
                for k in range(BLK)
            ]

        def blk(i, carry):
            base = s * (N_IDX_BLOCKS * SUB * BLK) + i * (SUB * BLK)
            pltpu.sync_copy(src_hbm.at[pl.ds(base, SUB * BLK)], srcbuf)
            pltpu.sync_copy(dst_hbm.at[pl.ds(base, SUB * BLK)], dstbuf)
            live = gathers(0, 0)
            for cp in live:
                cp.start()
            pend = []
            for j in range(SUB):
                for cp in live:
                    cp.wait()
                nxt = gathers((j + 1) % 2, j + 1) if j + 1 < SUB else []
                for cp in pend:
                    cp.wait()
                for cp in nxt:
                    cp.start()
                pend = [
                    pltpu.make_async_copy(
                        rows[j % 2].at[k],
                        acc.at[dstbuf.at[j * BLK + k]], sem2)
                    for k in range(BLK)
                ]
                for cp in pend:
                    cp.start(add=True)
                live = nxt
            for cp in pend:
                cp.wait()
            return carry

        lax.fori_loop(0, N_IDX_BLOCKS, blk, 0)
        plsc.subcore_barrier()

        def wchunk(q, carry):
            base = s * ROWS_PER_TILE + q * PWB
            pltpu.sync_copy(acc.at[pl.ds(base, PWB)], abuf)
            pltpu.sync_copy(slab_hbm.at[slab_idx].at[pl.ds(base, PWB)], tbuf)

            def wrow(i, carry2):
                abuf[i, :] = abuf[i, :] + tbuf[i, :]
                return carry2

            lax.fori_loop(0, PWB, wrow, 0)
            pltpu.sync_copy(abuf, out_hbm.at[slab_idx].at[pl.ds(base, PWB)])
            return carry

        lax.fori_loop(0, ROWS_PER_TILE // PWB, wchunk, 0)
        if r + 1 < n_rounds:
            plsc.subcore_barrier()


def _prop_call(slabs, src_r, dst_r, zeros, n_rounds):
    return pl.kernel(
        functools.partial(_prop_body, n_rounds),
        out_type=jax.ShapeDtypeStruct((2 * n_rounds, NP, 16), jnp.float32),
        mesh=_MESH,
        scratch_types=[
            pltpu.VMEM((SUB * BLK, CHUNK), jnp.int32),
            pltpu.VMEM((SUB * BLK, CHUNK), jnp.int32),
            pltpu.VMEM((BLK, CHUNK, 16), jnp.float32),
            pltpu.VMEM((BLK, CHUNK, 16), jnp.float32),
            pltpu.VMEM((PWB, 16), jnp.float32),
            pltpu.VMEM((PWB, 16), jnp.float32),
            pltpu.VMEM_SHARED((NP, 16), jnp.float32),
            pltpu.SemaphoreType.DMA,
            pltpu.SemaphoreType.DMA,
        ],
        compiler_params=_SC_PARAMS,
    )(slabs, src_r, dst_r, zeros)


def _prop_final_body(n_rounds, slab_hbm, src_hbm, dst_hbm, zeros_hbm,
                     dis_hbm, b2_hbm, out_hbm,
                     srcbuf, dstbuf, rows0, rows1, abuf, tbuf, dbuf, b2buf,
                     acc, sem, sem2):
    c = lax.axis_index("c")
    s = lax.axis_index("s")
    zsl = pl.ds(s * ROWS_PER_TILE, ROWS_PER_TILE)
    rows = (rows0, rows1)
    pltpu.sync_copy(b2_hbm, b2buf)
    for r in range(n_rounds):
        slab_idx = c * n_rounds + r
        pltpu.sync_copy(zeros_hbm, acc.at[zsl])
        plsc.subcore_barrier()

        def gathers(buf_id, j):
            return [
                pltpu.make_async_copy(
                    slab_hbm.at[slab_idx].at[srcbuf.at[j * BLK + k]],
                    rows[buf_id].at[k], sem)
                for k in range(BLK)
            ]

        def blk(i, carry):
            base = s * (N_IDX_BLOCKS * SUB * BLK) + i * (SUB * BLK)
            pltpu.sync_copy(src_hbm.at[pl.ds(base, SUB * BLK)], srcbuf)
            pltpu.sync_copy(dst_hbm.at[pl.ds(base, SUB * BLK)], dstbuf)
            live = gathers(0, 0)
            for cp in live:
                cp.start()
            pend = []
            for j in range(SUB):
                for cp in live:
                    cp.wait()
                nxt = gathers((j + 1) % 2, j + 1) if j + 1 < SUB else []
                for cp in pend:
                    cp.wait()
                for cp in nxt:
                    cp.start()
                pend = [
                    pltpu.make_async_copy(
                        rows[j % 2].at[k],
                        acc.at[dstbuf.at[j * BLK + k]], sem2)
                    for k in range(BLK)
                ]
                for cp in pend:
                    cp.start(add=True)
                live = nxt
            for cp in pend:
                cp.wait()
            return carry

        lax.fori_loop(0, N_IDX_BLOCKS, blk, 0)
        plsc.subcore_barrier()

        b2v = b2buf[slab_idx, :]

        def wchunk(q, carry):
            base = s * ROWS_PER_TILE + q * PWB
            pltpu.sync_copy(acc.at[pl.ds(base, PWB)], abuf)
            pltpu.sync_copy(slab_hbm.at[slab_idx].at[pl.ds(base, PWB)], tbuf)
            pltpu.sync_copy(dis_hbm.at[pl.ds(base, PWB)], dbuf)

            def wgrp(g, carry2):
                for l in range(16):
                    i = g * 16 + l
                    abuf[i, :] = (abuf[i, :] + tbuf[i, :]) * dbuf[i, :] + b2v
                return carry2

            lax.fori_loop(0, PWB // 16, wgrp, 0)
            pltpu.sync_copy(abuf, out_hbm.at[pl.ds(base, PWB),
                                             pl.ds(slab_idx * 16, 16)])
            return carry

        lax.fori_loop(0, ROWS_PER_TILE // PWB, wchunk, 0)
        if r + 1 < n_rounds:
            plsc.subcore_barrier()


def _prop_final_call(slabs, src_r, dst_r, zeros, dis, b2, n_rounds):
    return pl.kernel(
        functools.partial(_prop_final_body, n_rounds),
        out_type=jax.ShapeDtypeStruct((NP, OUT_DIM), jnp.float32),
        mesh=_MESH,
        scratch_types=[
            pltpu.VMEM((SUB * BLK, CHUNK), jnp.int32),
            pltpu.VMEM((SUB * BLK, CHUNK), jnp.int32),
            pltpu.VMEM((BLK, CHUNK, 16), jnp.float32),
            pltpu.VMEM((BLK, CHUNK, 16), jnp.float32),
            pltpu.VMEM((PWB, 16), jnp.float32),
            pltpu.VMEM((PWB, 16), jnp.float32),
            pltpu.VMEM((PWB, 16), jnp.float32),
            pltpu.VMEM((8, 16), jnp.float32),
            pltpu.VMEM_SHARED((NP, 16), jnp.float32),
            pltpu.SemaphoreType.DMA,
            pltpu.SemaphoreType.DMA,
        ],
        compiler_params=_SC_PARAMS,
    )(slabs, src_r, dst_r, zeros, dis, b2)


_TBLK = 1024
_TGRID = NP // _TBLK


def _dis_col(d):
    dT = d.T
    return jnp.concatenate([dT[:, r:r + 1] for r in range(8)], axis=0)


def _fold16(x):
    y = x.reshape(128, 8, 16)
    return jnp.concatenate([y[:, r, :] for r in range(8)], axis=-1)


def _unfold16(x):
    parts = [x[:, 16 * r:16 * (r + 1)] for r in range(8)]
    return jnp.stack(parts, axis=1).reshape(1024, 16)


def _scale_body(dego_ref, emb_ref, dis_ref, slab1_ref, disr_ref):
    deg = dego_ref[0] + dego_ref[1] + 1.0
    dis = lax.rsqrt(deg)
    dis_ref[...] = dis
    disT = _dis_col(dis)
    t1 = disT * emb_ref[...]
    slab1_ref[0] = _fold16(t1[:, :16])
    slab1_ref[1] = _fold16(t1[:, 16:])
    disr_ref[...] = _fold16(jnp.broadcast_to(disT, (_TBLK, 16)))


def _scale_call(dego, emb):
    return pl.pallas_call(
        _scale_body,
        grid=(_TGRID,),
        in_specs=[
            pl.BlockSpec((2, 8, 128), lambda i: (0, i, 0)),
            pl.BlockSpec((_TBLK, EMB_DIM), lambda i: (i, 0)),
        ],
        out_specs=[
            pl.BlockSpec((8, 128), lambda i: (i, 0)),
            pl.BlockSpec((2, 128, 128), lambda i: (0, i, 0)),
            pl.BlockSpec((128, 128), lambda i: (i, 0)),
        ],
        out_shape=[
            jax.ShapeDtypeStruct((NP // 128, 128), jnp.float32),
            jax.ShapeDtypeStruct((2, NPF, 128), jnp.float32),
            jax.ShapeDtypeStruct((NPF, 128), jnp.float32),
        ],
    )(dego, emb)


def _mid_body(dis_ref, u1_ref, W1_ref, b1_ref, W2_ref, slab2_ref):
    disT = _dis_col(dis_ref[...])
    u1 = jnp.concatenate([_unfold16(u1_ref[0]), _unfold16(u1_ref[1])],
                         axis=-1)
    g = disT * u1
    o1 = jnp.dot(g, W1_ref[...], preferred_element_type=jnp.float32) + b1_ref[...]
    h1 = jnp.maximum(o1, 0.0)
    t2 = disT * jnp.dot(h1, W2_ref[...], preferred_element_type=jnp.float32)
    for j in range(8):
        slab2_ref[j] = _fold16(t2[:, 16 * j:16 * (j + 1)])


def _mid_call(dis, u1s, W1, b1, W2):
    return pl.pallas_call(
        _mid_body,
        grid=(_TGRID,),
        in_specs=[
            pl.BlockSpec((8, 128), lambda i: (i, 0)),
            pl.BlockSpec((2, 128, 128), lambda i: (0, i, 0)),
            pl.BlockSpec((EMB_DIM, HIDDEN), lambda i: (0, 0)),
            pl.BlockSpec((1, HIDDEN), lambda i: (0, 0)),
            pl.BlockSpec((HIDDEN, OUT_DIM), lambda i: (0, 0)),
        ],
        out_specs=pl.BlockSpec((8, 128, 128), lambda i: (0, i, 0)),
        out_shape=jax.ShapeDtypeStruct((8, NPF, 128), jnp.float32),
    )(dis, u1s, W1, b1, W2)


def kernel(x, edge_index, emb, W1, b1, W2, b2):
    del x
    src = edge_index[0].astype(jnp.int32)
    dst = edge_index[1].astype(jnp.int32)
    pad = E_PAD - E
    src_r = jnp.concatenate(
        [src, jnp.full((pad,), N, jnp.int32)]).reshape(E_ROWS, CHUNK)
    dst_r = jnp.concatenate(
        [dst, jnp.full((pad,), N, jnp.int32)]).reshape(E_ROWS, CHUNK)
    ones = jnp.ones((CHUNK, 16), jnp.float32)
    zeros = jnp.zeros((ROWS_PER_TILE, 16), jnp.float32)

    dego = _deg_call(dst_r, ones, zeros)
    dis, slab1, disr = _scale_call(dego, emb)
    u1s = _prop_call(slab1.reshape(2, NP, 16), src_r, dst_r, zeros,
                     n_rounds=1)
    slab2 = _mid_call(dis, u1s.reshape(2, NPF, 128), W1,
                      b1.reshape(1, HIDDEN), W2)
    outv = _prop_final_call(slab2.reshape(8, NP, 16), src_r, dst_r, zeros,
                            disr.reshape(NP, 16), b2.reshape(8, 16),
                            n_rounds=4)
    return outv[:N]


# speedup vs baseline: 1.2600x; 1.0459x over previous
import functools

import jax
import jax.numpy as jnp
from jax import lax
from jax.experimental import pallas as pl
from jax.experimental.pallas import tpu as pltpu
from jax.experimental.pallas import tpu_sc as plsc

N = 100000
E = 1600000
EMB_DIM = 32
HIDDEN = 128
OUT_DIM = 128

NP = 100352
NPF = NP // 8
ROWS_PER_TILE = NP // 16
CROWS_PER_TILE = NP // 128 // 16
CHUNK = 128
BLK = 4
SUB = 7
N_IDX_BLOCKS = 28
E_PAD = 16 * N_IDX_BLOCKS * SUB * BLK * CHUNK
E_ROWS = E_PAD // CHUNK
DEG_BLK = 8
DEG_ROWS_PER_W = E_ROWS // 32
WB_CHUNK = 784
PWB = 128

_MESH = plsc.VectorSubcoreMesh(core_axis_name="c", subcore_axis_name="s")
_SC_PARAMS = pltpu.CompilerParams(use_tc_tiling_on_sc=False,
                                  needs_layout_passes=False)


def _deg_body(dst_hbm, ones_hbm, zeros_hbm, out_hbm,
              dstbuf, onesbuf, dbuf, cbuf, acc, sem):
    c = lax.axis_index("c")
    s = lax.axis_index("s")
    w = c * 16 + s
    pltpu.sync_copy(zeros_hbm, acc.at[pl.ds(s * ROWS_PER_TILE, ROWS_PER_TILE)])
    pltpu.sync_copy(ones_hbm, onesbuf)
    plsc.subcore_barrier()

    def blk(i, carry):
        base = w * DEG_ROWS_PER_W + i * DEG_BLK
        pltpu.sync_copy(dst_hbm.at[pl.ds(base, DEG_BLK)], dstbuf)
        cps = [
            pltpu.make_async_copy(onesbuf, acc.at[dstbuf.at[j]], sem)
            for j in range(DEG_BLK)
        ]
        for cp in cps:
            cp.start(add=True)
        for cp in cps:
            cp.wait()
        return carry

    lax.fori_loop(0, DEG_ROWS_PER_W // DEG_BLK, blk, 0)
    plsc.subcore_barrier()
    iota16 = lax.iota(jnp.int32, 16)
    zeros16 = jnp.zeros((16,), jnp.int32)

    def cchunk(q, carry):
        pltpu.sync_copy(acc.at[pl.ds(s * ROWS_PER_TILE + q * WB_CHUNK,
                                     WB_CHUNK)], dbuf)

        def cgrp(k, carry2):
            v = plsc.load_gather(dbuf, [iota16 + 16 * k, zeros16])
            g = q * (WB_CHUNK // 16) + k
            rowv = jnp.full((16,), g // 8, jnp.int32)
            colv = iota16 + (g % 8) * 16
            plsc.store_scatter(cbuf, [rowv, colv], v)
            return carry2

        lax.fori_loop(0, WB_CHUNK // 16, cgrp, 0)
        return carry

    lax.fori_loop(0, NP // 16 // WB_CHUNK, cchunk, 0)
    pltpu.sync_copy(cbuf, out_hbm.at[c].at[pl.ds(s * CROWS_PER_TILE,
                                                 CROWS_PER_TILE)])


def _deg_call(dst_r, ones, zeros):
    return pl.kernel(
        _deg_body,
        out_type=jax.ShapeDtypeStruct((2, NP // 128, 128), jnp.float32),
        mesh=_MESH,
        scratch_types=[
            pltpu.VMEM((DEG_BLK, CHUNK), jnp.int32),
            pltpu.VMEM((CHUNK, 16), jnp.float32),
            pltpu.VMEM((WB_CHUNK, 16), jnp.float32),
            pltpu.VMEM((CROWS_PER_TILE, 128), jnp.float32),
            pltpu.VMEM_SHARED((NP, 16), jnp.float32),
            pltpu.SemaphoreType.DMA,
        ],
        compiler_params=_SC_PARAMS,
    )(dst_r, ones, zeros)


def _prop_body(n_rounds, slab_hbm, src_hbm, dst_hbm, zeros_hbm, out_hbm,
               srcbuf, dstbuf, rows0, rows1, abuf, tbuf, acc, sem, sem2):
    c = lax.axis_index("c")
    s = lax.axis_index("s")
    zsl = pl.ds(s * ROWS_PER_TILE, ROWS_PER_TILE)
    rows = (rows0, rows1)
    for r in range(n_rounds):
        slab_idx = c * n_rounds + r
        pltpu.sync_copy(zeros_hbm, acc.at[zsl])
        plsc.subcore_barrier()

        def gathers(buf_id, j):
            return [
                pltpu.make_async_copy(
                    slab_hbm.at[slab_idx].at[srcbuf.at[j * BLK + k]],
                    rows[buf_id].at[k], sem)
                for k in range(BLK)
            ]

        def blk(i, carry):
            base = s * (N_IDX_BLOCKS * SUB * BLK) + i * (SUB * BLK)
            pltpu.sync_copy(src_hbm.at[pl.ds(base, SUB * BLK)], srcbuf)
            pltpu.sync_copy(dst_hbm.at[pl.ds(base, SUB * BLK)], dstbuf)
            live = gathers(0, 0)
            for cp in live:
                cp.start()
            pend = []
            for j in range(SUB):
                for cp in live:
                    cp.wait()
                nxt = gathers((j + 1) % 2, j + 1) if j + 1 < SUB else []
                for cp in pend:
                    cp.wait()
                for cp in nxt:
                    cp.start()
                pend = [
                    pltpu.make_async_copy(
                        rows[j % 2].at[k],
                        acc.at[dstbuf.at[j * BLK + k]], sem2)
                    for k in range(BLK)
                ]
                for cp in pend:
                    cp.start(add=True)
                live = nxt
            for cp in pend:
                cp.wait()
            return carry

        lax.fori_loop(0, N_IDX_BLOCKS, blk, 0)
        plsc.subcore_barrier()

        def wchunk(q, carry):
            base = s * ROWS_PER_TILE + q * PWB
            pltpu.sync_copy(acc.at[pl.ds(base, PWB)], abuf)
            pltpu.sync_copy(slab_hbm.at[slab_idx].at[pl.ds(base, PWB)], tbuf)

            def wrow(i, carry2):
                abuf[i, :] = abuf[i, :] + tbuf[i, :]
                return carry2

            lax.fori_loop(0, PWB, wrow, 0)
            pltpu.sync_copy(abuf, out_hbm.at[pl.ds(base, PWB),
                                             pl.ds(slab_idx * 16, 16)])
            return carry

        lax.fori_loop(0, ROWS_PER_TILE // PWB, wchunk, 0)
        if r + 1 < n_rounds:
            plsc.subcore_barrier()


def _prop_call(slabs, src_r, dst_r, zeros, n_rounds):
    return pl.kernel(
        functools.partial(_prop_body, n_rounds),
        out_type=jax.ShapeDtypeStruct((NP, 2 * n_rounds * 16), jnp.float32),
        mesh=_MESH,
        scratch_types=[
            pltpu.VMEM((SUB * BLK, CHUNK), jnp.int32),
            pltpu.VMEM((SUB * BLK, CHUNK), jnp.int32),
            pltpu.VMEM((BLK, CHUNK, 16), jnp.float32),
            pltpu.VMEM((BLK, CHUNK, 16), jnp.float32),
            pltpu.VMEM((PWB, 16), jnp.float32),
            pltpu.VMEM((PWB, 16), jnp.float32),
            pltpu.VMEM_SHARED((NP, 16), jnp.float32),
            pltpu.SemaphoreType.DMA,
            pltpu.SemaphoreType.DMA,
        ],
        compiler_params=_SC_PARAMS,
    )(slabs, src_r, dst_r, zeros)


def _prop_final_body(n_rounds, slab_hbm, src_hbm, dst_hbm, zeros_hbm,
                     dis_hbm, b2_hbm, out_hbm,
                     srcbuf, dstbuf, rows0, rows1, abuf, tbuf, dbuf, b2buf,
                     acc, sem, sem2):
    c = lax.axis_index("c")
    s = lax.axis_index("s")
    zsl = pl.ds(s * ROWS_PER_TILE, ROWS_PER_TILE)
    rows = (rows0, rows1)
    pltpu.sync_copy(b2_hbm, b2buf)
    for r in range(n_rounds):
        slab_idx = c * n_rounds + r
        pltpu.sync_copy(zeros_hbm, acc.at[zsl])
        plsc.subcore_barrier()

        def gathers(buf_id, j):
            return [
                pltpu.make_async_copy(
                    slab_hbm.at[slab_idx].at[srcbuf.at[j * BLK + k]],
                    rows[buf_id].at[k], sem)
                for k in range(BLK)
            ]

        def blk(i, carry):
            base = s * (N_IDX_BLOCKS * SUB * BLK) + i * (SUB * BLK)
            pltpu.sync_copy(src_hbm.at[pl.ds(base, SUB * BLK)], srcbuf)
            pltpu.sync_copy(dst_hbm.at[pl.ds(base, SUB * BLK)], dstbuf)
            live = gathers(0, 0)
            for cp in live:
                cp.start()
            pend = []
            for j in range(SUB):
                for cp in live:
                    cp.wait()
                nxt = gathers((j + 1) % 2, j + 1) if j + 1 < SUB else []
                for cp in pend:
                    cp.wait()
                for cp in nxt:
                    cp.start()
                pend = [
                    pltpu.make_async_copy(
                        rows[j % 2].at[k],
                        acc.at[dstbuf.at[j * BLK + k]], sem2)
                    for k in range(BLK)
                ]
                for cp in pend:
                    cp.start(add=True)
                live = nxt
            for cp in pend:
                cp.wait()
            return carry

        lax.fori_loop(0, N_IDX_BLOCKS, blk, 0)
        plsc.subcore_barrier()

        b2v = b2buf[slab_idx, :]

        def wchunk(q, carry):
            base = s * ROWS_PER_TILE + q * PWB
            pltpu.sync_copy(acc.at[pl.ds(base, PWB)], abuf)
            pltpu.sync_copy(slab_hbm.at[slab_idx].at[pl.ds(base, PWB)], tbuf)
            pltpu.sync_copy(dis_hbm.at[pl.ds(base, PWB)], dbuf)

            def wgrp(g, carry2):
                for l in range(16):
                    i = g * 16 + l
                    abuf[i, :] = (abuf[i, :] + tbuf[i, :]) * dbuf[i, :] + b2v
                return carry2

            lax.fori_loop(0, PWB // 16, wgrp, 0)
            pltpu.sync_copy(abuf, out_hbm.at[pl.ds(base, PWB),
                                             pl.ds(slab_idx * 16, 16)])
            return carry

        lax.fori_loop(0, ROWS_PER_TILE // PWB, wchunk, 0)
        if r + 1 < n_rounds:
            plsc.subcore_barrier()


def _prop_final_call(slabs, src_r, dst_r, zeros, dis, b2, n_rounds):
    return pl.kernel(
        functools.partial(_prop_final_body, n_rounds),
        out_type=jax.ShapeDtypeStruct((NP, OUT_DIM), jnp.float32),
        mesh=_MESH,
        scratch_types=[
            pltpu.VMEM((SUB * BLK, CHUNK), jnp.int32),
            pltpu.VMEM((SUB * BLK, CHUNK), jnp.int32),
            pltpu.VMEM((BLK, CHUNK, 16), jnp.float32),
            pltpu.VMEM((BLK, CHUNK, 16), jnp.float32),
            pltpu.VMEM((PWB, 16), jnp.float32),
            pltpu.VMEM((PWB, 16), jnp.float32),
            pltpu.VMEM((PWB, 16), jnp.float32),
            pltpu.VMEM((8, 16), jnp.float32),
            pltpu.VMEM_SHARED((NP, 16), jnp.float32),
            pltpu.SemaphoreType.DMA,
            pltpu.SemaphoreType.DMA,
        ],
        compiler_params=_SC_PARAMS,
    )(slabs, src_r, dst_r, zeros, dis, b2)


_TBLK = 1024
_TGRID = NP // _TBLK


def _dis_col(d):
    dT = d.T
    return jnp.concatenate([dT[:, r:r + 1] for r in range(8)], axis=0)


def _fold16(x):
    y = x.reshape(128, 8, 16)
    return jnp.concatenate([y[:, r, :] for r in range(8)], axis=-1)


def _unfold16(x):
    parts = [x[:, 16 * r:16 * (r + 1)] for r in range(8)]
    return jnp.stack(parts, axis=1).reshape(1024, 16)


def _scale_body(dego_ref, emb_ref, dis_ref, slab1_ref, disr_ref):
    deg = dego_ref[0] + dego_ref[1] + 1.0
    dis = lax.rsqrt(deg)
    dis_ref[...] = dis
    disT = _dis_col(dis)
    t1 = disT * emb_ref[...]
    slab1_ref[0] = _fold16(t1[:, :16])
    slab1_ref[1] = _fold16(t1[:, 16:])
    disr_ref[...] = _fold16(jnp.broadcast_to(disT, (_TBLK, 16)))


def _scale_call(dego, emb):
    return pl.pallas_call(
        _scale_body,
        grid=(_TGRID,),
        in_specs=[
            pl.BlockSpec((2, 8, 128), lambda i: (0, i, 0)),
            pl.BlockSpec((_TBLK, EMB_DIM), lambda i: (i, 0)),
        ],
        out_specs=[
            pl.BlockSpec((8, 128), lambda i: (i, 0)),
            pl.BlockSpec((2, 128, 128), lambda i: (0, i, 0)),
            pl.BlockSpec((128, 128), lambda i: (i, 0)),
        ],
        out_shape=[
            jax.ShapeDtypeStruct((NP // 128, 128), jnp.float32),
            jax.ShapeDtypeStruct((2, NPF, 128), jnp.float32),
            jax.ShapeDtypeStruct((NPF, 128), jnp.float32),
        ],
    )(dego, emb)


def _mid_body(dis_ref, u1_ref, W1_ref, b1_ref, W2_ref, slab2_ref):
    disT = _dis_col(dis_ref[...])
    u = u1_ref[...]
    parts = [u[:, 32 * r:32 * (r + 1)] for r in range(4)]
    u1 = jnp.stack(parts, axis=1).reshape(1024, 32)
    g = disT * u1
    o1 = jnp.dot(g, W1_ref[...], preferred_element_type=jnp.float32) + b1_ref[...]
    h1 = jnp.maximum(o1, 0.0)
    t2 = disT * jnp.dot(h1, W2_ref[...], preferred_element_type=jnp.float32)
    for j in range(8):
        slab2_ref[j] = _fold16(t2[:, 16 * j:16 * (j + 1)])


def _mid_call(dis, u1s, W1, b1, W2):
    return pl.pallas_call(
        _mid_body,
        grid=(_TGRID,),
        in_specs=[
            pl.BlockSpec((8, 128), lambda i: (i, 0)),
            pl.BlockSpec((256, 128), lambda i: (i, 0)),
            pl.BlockSpec((EMB_DIM, HIDDEN), lambda i: (0, 0)),
            pl.BlockSpec((1, HIDDEN), lambda i: (0, 0)),
            pl.BlockSpec((HIDDEN, OUT_DIM), lambda i: (0, 0)),
        ],
        out_specs=pl.BlockSpec((8, 128, 128), lambda i: (0, i, 0)),
        out_shape=jax.ShapeDtypeStruct((8, NPF, 128), jnp.float32),
    )(dis, u1s, W1, b1, W2)


def kernel(x, edge_index, emb, W1, b1, W2, b2):
    del x
    src = edge_index[0].astype(jnp.int32)
    dst = edge_index[1].astype(jnp.int32)
    pad = E_PAD - E
    src_r = jnp.concatenate(
        [src, jnp.full((pad,), N, jnp.int32)]).reshape(E_ROWS, CHUNK)
    dst_r = jnp.concatenate(
        [dst, jnp.full((pad,), N, jnp.int32)]).reshape(E_ROWS, CHUNK)
    ones = jnp.ones((CHUNK, 16), jnp.float32)
    zeros = jnp.zeros((ROWS_PER_TILE, 16), jnp.float32)

    dego = _deg_call(dst_r, ones, zeros)
    dis, slab1, disr = _scale_call(dego, emb)
    u1s = _prop_call(slab1.reshape(2, NP, 16), src_r, dst_r, zeros,
                     n_rounds=1)
    slab2 = _mid_call(dis, u1s.reshape(NP // 4, 128), W1,
                      b1.reshape(1, HIDDEN), W2)
    outv = _prop_final_call(slab2.reshape(8, NP, 16), src_r, dst_r, zeros,
                            disr.reshape(NP, 16), b2.reshape(8, 16),
                            n_rounds=4)
    return outv[:N]
